# baseline (speedup 1.0000x reference)
import functools

import jax
import jax.numpy as jnp
from jax.experimental import pallas as pl

K = 1024
C = 64
BN = 512


def _vq_block(xf_ref, cb_ref, quant_ref, idx_ref, loss_ref):
    x = xf_ref[...]
    cb = cb_ref[...]
    cross = jax.lax.dot_general(
        x, cb, (((1,), (1,)), ((), ())),
        preferred_element_type=jnp.float32)
    x_sq = jnp.sum(x * x, axis=-1, keepdims=True)
    e_sq = jnp.sum(cb * cb, axis=-1)
    d2 = x_sq + e_sq[None, :] - 2.0 * cross
    m = jnp.min(d2, axis=-1, keepdims=True)
    iota = jax.lax.broadcasted_iota(jnp.int32, d2.shape, 1)
    idx = jnp.min(jnp.where(d2 == m, iota, K), axis=-1)
    onehot = (iota == idx[:, None]).astype(jnp.float32)
    quant = jax.lax.dot_general(
        onehot, cb, (((1,), (0,)), ((), ())),
        preferred_element_type=jnp.float32,
        precision=jax.lax.Precision.HIGHEST)
    quant_ref[...] = x + (quant - x)
    idx_ref[0, 0, :] = idx

    @pl.when(pl.program_id(0) == 0)
    def _init():
        loss_ref[...] = jnp.zeros((1, 1), jnp.float32)

    loss_ref[...] += jnp.sum((x - quant) ** 2).reshape(1, 1)


@functools.partial(jax.jit, static_argnames=())
def kernel(x, codebook):
    B, Cc, H, W = x.shape
    N = B * H * W
    xf = jnp.transpose(x, (0, 2, 3, 1)).reshape(N, Cc)
    grid = N // BN
    quant, idx3, loss2 = pl.pallas_call(
        _vq_block,
        grid=(grid,),
        in_specs=[
            pl.BlockSpec((BN, C), lambda i: (i, 0)),
            pl.BlockSpec((K, C), lambda i: (0, 0)),
        ],
        out_specs=[
            pl.BlockSpec((BN, C), lambda i: (i, 0)),
            pl.BlockSpec((1, 1, BN), lambda i: (i, 0, 0)),
            pl.BlockSpec((1, 1), lambda i: (0, 0)),
        ],
        out_shape=[
            jax.ShapeDtypeStruct((N, C), jnp.float32),
            jax.ShapeDtypeStruct((grid, 1, BN), jnp.int32),
            jax.ShapeDtypeStruct((1, 1), jnp.float32),
        ],
    )(xf, codebook)
    loss = (loss2[0, 0] / (N * C)).astype(jnp.float32)
    quant_out = jnp.transpose(quant.reshape(B, H, W, Cc), (0, 3, 1, 2))
    idx_emb = idx3.reshape(B, H * W)
    return (quant_out, loss, loss, idx_emb)


# speedup vs baseline: 1.0633x; 1.0633x over previous
import functools

import jax
import jax.numpy as jnp
from jax import lax
from jax.experimental import pallas as pl
from jax.experimental.pallas import tpu as pltpu
from jax.experimental.pallas import tpu_sc as plsc

K = 1024
C = 64
BN = 512


def _vq_dist_block(xf_ref, cb_ref, idx_ref, loss_ref):
    x = xf_ref[...]
    cb = cb_ref[...]
    cross = jax.lax.dot_general(
        x, cb, (((1,), (1,)), ((), ())),
        preferred_element_type=jnp.float32)
    x_sq = jnp.sum(x * x, axis=-1, keepdims=True)
    e_sq = jnp.sum(cb * cb, axis=-1)
    d2 = x_sq + e_sq[None, :] - 2.0 * cross
    m = jnp.min(d2, axis=-1, keepdims=True)
    iota = jax.lax.broadcasted_iota(jnp.int32, d2.shape, 1)
    idx = jnp.min(jnp.where(d2 == m, iota, K), axis=-1)
    idx_ref[0, 0, :] = idx

    @pl.when(pl.program_id(0) == 0)
    def _init():
        loss_ref[...] = jnp.zeros((1, 1), jnp.float32)

    loss_ref[...] += jnp.sum(m).reshape(1, 1)


def _make_sc_gather(B):
    info = plsc.get_sparse_core_info()
    NW = info.num_cores * info.num_subcores
    b_per_w = B // NW
    mesh = plsc.VectorSubcoreMesh(core_axis_name="c", subcore_axis_name="s")

    @functools.partial(
        pl.kernel, mesh=mesh,
        compiler_params=pltpu.CompilerParams(use_tc_tiling_on_sc=False),
        out_type=jax.ShapeDtypeStruct((B, C), jnp.float32),
        scratch_types=[
            pltpu.VMEM((b_per_w,), jnp.int32),
            pltpu.VMEM((b_per_w, C), jnp.float32),
            pltpu.SemaphoreType.DMA,
        ],
    )
    def gather(table_hbm, idx_hbm, out_hbm, idx_v, rows_v, sem):
        wid = lax.axis_index("s") * info.num_cores + lax.axis_index("c")
        base = wid * b_per_w
        pltpu.sync_copy(idx_hbm.at[pl.ds(base, b_per_w)], idx_v)
        pltpu.async_copy(table_hbm.at[idx_v], rows_v, sem).wait()
        pltpu.sync_copy(rows_v, out_hbm.at[pl.ds(base, b_per_w)])

    return gather


@jax.jit
def kernel(x, codebook):
    B, Cc, H, W = x.shape
    N = B * H * W
    xf = jnp.transpose(x, (0, 2, 3, 1)).reshape(N, Cc)
    grid = N // BN
    idx3, loss2 = pl.pallas_call(
        _vq_dist_block,
        grid=(grid,),
        in_specs=[
            pl.BlockSpec((BN, C), lambda i: (i, 0)),
            pl.BlockSpec((K, C), lambda i: (0, 0)),
        ],
        out_specs=[
            pl.BlockSpec((1, 1, BN), lambda i: (i, 0, 0)),
            pl.BlockSpec((1, 1), lambda i: (0, 0)),
        ],
        out_shape=[
            jax.ShapeDtypeStruct((grid, 1, BN), jnp.int32),
            jax.ShapeDtypeStruct((1, 1), jnp.float32),
        ],
    )(xf, codebook)
    idx_flat = idx3.reshape(N)
    quant = _make_sc_gather(N)(codebook, idx_flat)
    loss = (loss2[0, 0] / (N * Cc)).astype(jnp.float32)
    quant_out = jnp.transpose(quant.reshape(B, H, W, Cc), (0, 3, 1, 2))
    idx_emb = idx_flat.reshape(B, H * W)
    return (quant_out, loss, loss, idx_emb)
